# Initial kernel scaffold; baseline (speedup 1.0000x reference)
import jax
import jax.numpy as jnp
from jax.experimental import pallas as pl


def kernel(feature, edge_index, W, a_src, a_dst, bias):
    raise NotImplementedError("write your pallas kernel here")


# speedup vs baseline: 56.6533x; 56.6533x over previous
import functools

import jax
import jax.numpy as jnp
from jax import lax
from jax.experimental import pallas as pl
from jax.experimental.pallas import tpu as pltpu
from jax.experimental.pallas import tpu_sc as plsc

N = 10000
E = 320000
IN_DIM = 128
NH = 8
FO = 16
DH = NH * FO
DT = DH + 16

NC = 2
NS = 16
NW = NC * NS
EPW = E // NW
CHUNK = 80
NCHUNK = EPW // CHUNK
NP = 10240
RPT = NP // NS

ROW_BLK = 2000
GRID = N // ROW_BLK


def _proj_body(x_ref, w144_ref, w16_ref, hext_ref, ud_ref):
    x = x_ref[...]
    hext_ref[...] = jnp.dot(x, w144_ref[...], preferred_element_type=jnp.float32)
    ud_ref[...] = jnp.dot(x, w16_ref[...], preferred_element_type=jnp.float32)


def _project(feature, w144, w16):
    return pl.pallas_call(
        _proj_body,
        grid=(GRID,),
        in_specs=[
            pl.BlockSpec((ROW_BLK, IN_DIM), lambda i: (i, 0)),
            pl.BlockSpec((IN_DIM, DT), lambda i: (0, 0)),
            pl.BlockSpec((IN_DIM, 16), lambda i: (0, 0)),
        ],
        out_specs=[
            pl.BlockSpec((ROW_BLK, DT), lambda i: (i, 0)),
            pl.BlockSpec((ROW_BLK, 16), lambda i: (i, 0)),
        ],
        out_shape=[
            jax.ShapeDtypeStruct((N, DT), jnp.float32),
            jax.ShapeDtypeStruct((N, 16), jnp.float32),
        ],
    )(feature, w144, w16)


def _sc_edges_body(hext_hbm, ud_hbm, src_hbm, dst_hbm, cvec_hbm, acc_hbm,
                   idxs_v, idxd_v, hrow_v, udr_v, msg_v, ex_v, cv_v,
                   acc_sh, sem):
    cid = lax.axis_index("c")
    sid = lax.axis_index("s")
    wid = sid * NC + cid

    zero16 = jnp.zeros((16,), jnp.float32)

    def zrow(r, carry):
        for k in range(DT // 16):
            msg_v[r, pl.ds(16 * k, 16)] = zero16
        return carry

    lax.fori_loop(0, CHUNK, zrow, 0)

    def zcopy(j, carry):
        pltpu.sync_copy(msg_v, acc_sh.at[pl.ds(sid * RPT + j * CHUNK, CHUNK)])
        return carry

    lax.fori_loop(0, RPT // CHUNK, zcopy, 0)
    plsc.subcore_barrier()

    pltpu.sync_copy(cvec_hbm, cv_v)
    cv = cv_v[...]

    def chunk(ci, carry):
        base = wid * EPW + ci * CHUNK
        pltpu.sync_copy(src_hbm.at[pl.ds(base, CHUNK)], idxs_v)
        pltpu.sync_copy(dst_hbm.at[pl.ds(base, CHUNK)], idxd_v)
        cp1 = pltpu.async_copy(hext_hbm.at[idxs_v], hrow_v, sem)
        cp2 = pltpu.async_copy(ud_hbm.at[idxd_v], udr_v, sem)
        cp1.wait()
        cp2.wait()

        lane = lax.iota(jnp.int32, 16)

        def edge(e, ecarry):
            s16 = hrow_v[e, pl.ds(DH, 16)] + udr_v[e, :]
            l16 = jnp.maximum(s16, 0.2 * s16)
            ex = jnp.exp(l16 - cv)
            msg_v[e, pl.ds(DH, 16)] = ex
            for k in range(NH):
                onehot = (lane == k).astype(jnp.float32)
                bk = jnp.full((16,), jnp.sum(ex * onehot))
                msg_v[e, pl.ds(16 * k, 16)] = hrow_v[e, pl.ds(16 * k, 16)] * bk
            return ecarry

        lax.fori_loop(0, CHUNK, edge, 0)
        pltpu.sync_copy(msg_v, acc_sh.at[idxd_v], add=True)
        return carry

    lax.fori_loop(0, NCHUNK, chunk, 0)
    plsc.subcore_barrier()

    pltpu.sync_copy(acc_sh.at[pl.ds(sid * RPT, RPT)],
                    acc_hbm.at[cid, pl.ds(sid * RPT, RPT)])


def _sc_edges(hext, ud, src, dst, cvec):
    fn = pl.kernel(
        _sc_edges_body,
        out_type=jax.ShapeDtypeStruct((NC, NP, DT), jnp.float32),
        mesh=plsc.VectorSubcoreMesh(core_axis_name="c", subcore_axis_name="s"),
        scratch_types=[
            pltpu.VMEM((CHUNK,), jnp.int32),
            pltpu.VMEM((CHUNK,), jnp.int32),
            pltpu.VMEM((CHUNK, DT), jnp.float32),
            pltpu.VMEM((CHUNK, 16), jnp.float32),
            pltpu.VMEM((CHUNK, DT), jnp.float32),
            pltpu.VMEM((16,), jnp.float32),
            pltpu.VMEM((16,), jnp.float32),
            pltpu.VMEM_SHARED((NP, DT), jnp.float32),
            pltpu.SemaphoreType.DMA,
        ],
        compiler_params=pltpu.CompilerParams(
            needs_layout_passes=False, use_tc_tiling_on_sc=False),
    )
    return fn(hext, ud, src, dst, cvec)


def _epi_body(a0_ref, a1_ref, f_ref, b_ref, e_ref, o_ref):
    s = a0_ref[0] + a1_ref[0]
    num = lax.slice(s, (0, 0), (ROW_BLK, DH))
    den = lax.slice(s, (0, DH), (ROW_BLK, DT))
    den_exp = jnp.dot(den, e_ref[...], preferred_element_type=jnp.float32)
    val = num / (den_exp + 1e-16) + b_ref[...]
    o_ref[...] = f_ref[...] + jnp.where(val > 0, val, jnp.exp(val) - 1.0)


def _epilogue(acc, feature, bias_row, eexp):
    return pl.pallas_call(
        _epi_body,
        grid=(GRID,),
        in_specs=[
            pl.BlockSpec((1, ROW_BLK, DT), lambda i: (0, i, 0)),
            pl.BlockSpec((1, ROW_BLK, DT), lambda i: (1, i, 0)),
            pl.BlockSpec((ROW_BLK, DH), lambda i: (i, 0)),
            pl.BlockSpec((1, DH), lambda i: (0, 0)),
            pl.BlockSpec((16, DH), lambda i: (0, 0)),
        ],
        out_specs=pl.BlockSpec((ROW_BLK, DH), lambda i: (i, 0)),
        out_shape=jax.ShapeDtypeStruct((N, DH), jnp.float32),
    )(acc, acc, feature, bias_row, eexp)


def kernel(feature, edge_index, W, a_src, a_dst, bias):
    Wr = W.reshape(IN_DIM, NH, FO)
    Ws = jnp.sum(Wr * a_src[None], axis=-1)
    Wd = jnp.sum(Wr * a_dst[None], axis=-1)
    pad8 = jnp.zeros((IN_DIM, 8), jnp.float32)
    w144 = jnp.concatenate([W, Ws, pad8], axis=1)
    w16 = jnp.concatenate([Wd, pad8], axis=1)

    hext, ud = _project(feature, w144, w16)

    cs = jnp.max(lax.slice(hext, (0, DH), (N, DH + NH)))
    cd = jnp.max(lax.slice(ud, (0, 0), (N, NH)))
    csum = cs + cd
    c = jnp.maximum(csum, 0.2 * csum)
    cvec = jnp.full((16,), c, jnp.float32)

    src = edge_index[0]
    dst = edge_index[1]
    acc = _sc_edges(hext, ud, src, dst, cvec)
    acc = lax.slice(acc, (0, 0, 0), (NC, N, DT))

    eexp = jnp.repeat(jnp.eye(NH, dtype=jnp.float32), FO, axis=1)
    eexp = jnp.concatenate([eexp, jnp.zeros((16 - NH, DH), jnp.float32)], axis=0)
    bias_row = bias.reshape(1, DH)
    return _epilogue(acc, feature, bias_row, eexp)


# speedup vs baseline: 71.6326x; 1.2644x over previous
import functools

import jax
import jax.numpy as jnp
from jax import lax
from jax.experimental import pallas as pl
from jax.experimental.pallas import tpu as pltpu
from jax.experimental.pallas import tpu_sc as plsc

N = 10000
E = 320000
IN_DIM = 128
NH = 8
FO = 16
DH = NH * FO
DT = DH + 16

NC = 2
NS = 16
NW = NC * NS
EPW = E // NW
CHUNK = 40
NCHUNK = EPW // CHUNK
NPAIR = NCHUNK // 2
NP = 10240
RPT = NP // NS

ROW_BLK = 2000
GRID = N // ROW_BLK


def _proj_body(x_ref, w144_ref, w16_ref, hext_ref, ud_ref):
    x = x_ref[...]
    hext_ref[...] = jnp.dot(x, w144_ref[...], preferred_element_type=jnp.float32)
    ud_ref[...] = jnp.dot(x, w16_ref[...], preferred_element_type=jnp.float32)


def _project(feature, w144, w16):
    return pl.pallas_call(
        _proj_body,
        grid=(GRID,),
        in_specs=[
            pl.BlockSpec((ROW_BLK, IN_DIM), lambda i: (i, 0)),
            pl.BlockSpec((IN_DIM, DT), lambda i: (0, 0)),
            pl.BlockSpec((IN_DIM, 16), lambda i: (0, 0)),
        ],
        out_specs=[
            pl.BlockSpec((ROW_BLK, DT), lambda i: (i, 0)),
            pl.BlockSpec((ROW_BLK, 16), lambda i: (i, 0)),
        ],
        out_shape=[
            jax.ShapeDtypeStruct((N, DT), jnp.float32),
            jax.ShapeDtypeStruct((N, 16), jnp.float32),
        ],
    )(feature, w144, w16)


def _sc_edges_body(hext_hbm, ud_hbm, src_hbm, dst_hbm, cvec_hbm, acc_hbm,
                   idxs_v, idxd_v, hrow_v, udr_v, msg_v, cv_v,
                   acc_sh, semi0, semi1, semg0, semg1):
    cid = lax.axis_index("c")
    sid = lax.axis_index("s")
    wid = sid * NC + cid
    semi = (semi0, semi1)
    semg = (semg0, semg1)

    zero16 = jnp.zeros((16,), jnp.float32)

    def zrow(r, carry):
        for pp in range(2):
            for k in range(DT // 16):
                msg_v[pp, r, pl.ds(16 * k, 16)] = zero16
        return carry

    lax.fori_loop(0, CHUNK, zrow, 0)

    def zcopy(j, carry):
        pltpu.sync_copy(msg_v.at[0], acc_sh.at[pl.ds(sid * RPT + j * CHUNK, CHUNK)])
        return carry

    lax.fori_loop(0, RPT // CHUNK, zcopy, 0)
    plsc.subcore_barrier()

    pltpu.sync_copy(cvec_hbm, cv_v)
    cv = cv_v[...]

    def idx_start(x, p):
        base = wid * EPW + x * CHUNK
        pltpu.async_copy(src_hbm.at[pl.ds(base, CHUNK)], idxs_v.at[p], semi[p])
        pltpu.async_copy(dst_hbm.at[pl.ds(base, CHUNK)], idxd_v.at[p], semi[p])

    def idx_wait(p):
        pltpu.make_async_copy(src_hbm.at[pl.ds(0, CHUNK)], idxs_v.at[p], semi[p]).wait()
        pltpu.make_async_copy(dst_hbm.at[pl.ds(0, CHUNK)], idxd_v.at[p], semi[p]).wait()

    def gath_start(p):
        pltpu.async_copy(hext_hbm.at[idxs_v.at[p]], hrow_v.at[p], semg[p])
        pltpu.async_copy(ud_hbm.at[idxd_v.at[p]], udr_v.at[p], semg[p])

    def gath_wait(p):
        pltpu.make_async_copy(hext_hbm.at[idxs_v.at[p]], hrow_v.at[p], semg[p]).wait()
        pltpu.make_async_copy(ud_hbm.at[idxd_v.at[p]], udr_v.at[p], semg[p]).wait()

    lane = lax.iota(jnp.int32, 16)

    def compute(p):
        def edge2(j, ecarry):
            for t in range(2):
                e = 2 * j + t
                s16 = hrow_v[p, e, pl.ds(DH, 16)] + udr_v[p, e, :]
                l16 = jnp.maximum(s16, 0.2 * s16)
                ex = jnp.exp(l16 - cv)
                msg_v[p, e, pl.ds(DH, 16)] = ex
                for k in range(NH):
                    onehot = (lane == k).astype(jnp.float32)
                    bk = jnp.full((16,), jnp.sum(ex * onehot))
                    msg_v[p, e, pl.ds(16 * k, 16)] = (
                        hrow_v[p, e, pl.ds(16 * k, 16)] * bk)
            return ecarry

        lax.fori_loop(0, CHUNK // 2, edge2, 0)

    def half(x, p, po):
        idx_wait(po)
        gath_start(po)
        gath_wait(p)
        compute(p)
        pltpu.sync_copy(msg_v.at[p], acc_sh.at[idxd_v.at[p]], add=True)
        idx_start(jnp.minimum(x + 2, NCHUNK - 1), p)

    idx_start(0, 0)
    idx_wait(0)
    gath_start(0)
    idx_start(1, 1)

    def pair(i, carry):
        half(2 * i, 0, 1)
        half(2 * i + 1, 1, 0)
        return carry

    lax.fori_loop(0, NPAIR, pair, 0)
    idx_wait(1)
    gath_wait(0)
    plsc.subcore_barrier()

    pltpu.sync_copy(acc_sh.at[pl.ds(sid * RPT, RPT)],
                    acc_hbm.at[cid, pl.ds(sid * RPT, RPT)])


def _sc_edges(hext, ud, src, dst, cvec):
    fn = pl.kernel(
        _sc_edges_body,
        out_type=jax.ShapeDtypeStruct((NC, NP, DT), jnp.float32),
        mesh=plsc.VectorSubcoreMesh(core_axis_name="c", subcore_axis_name="s"),
        scratch_types=[
            pltpu.VMEM((2, CHUNK), jnp.int32),
            pltpu.VMEM((2, CHUNK), jnp.int32),
            pltpu.VMEM((2, CHUNK, DT), jnp.float32),
            pltpu.VMEM((2, CHUNK, 16), jnp.float32),
            pltpu.VMEM((2, CHUNK, DT), jnp.float32),
            pltpu.VMEM((16,), jnp.float32),
            pltpu.VMEM_SHARED((NP, DT), jnp.float32),
            pltpu.SemaphoreType.DMA,
            pltpu.SemaphoreType.DMA,
            pltpu.SemaphoreType.DMA,
            pltpu.SemaphoreType.DMA,
        ],
        compiler_params=pltpu.CompilerParams(
            needs_layout_passes=False, use_tc_tiling_on_sc=False),
    )
    return fn(hext, ud, src, dst, cvec)


def _epi_body(a0_ref, a1_ref, f_ref, b_ref, e_ref, o_ref):
    s = a0_ref[0] + a1_ref[0]
    num = lax.slice(s, (0, 0), (ROW_BLK, DH))
    den = lax.slice(s, (0, DH), (ROW_BLK, DT))
    den_exp = jnp.dot(den, e_ref[...], preferred_element_type=jnp.float32)
    val = num / (den_exp + 1e-16) + b_ref[...]
    o_ref[...] = f_ref[...] + jnp.where(val > 0, val, jnp.exp(val) - 1.0)


def _epilogue(acc, feature, bias_row, eexp):
    return pl.pallas_call(
        _epi_body,
        grid=(GRID,),
        in_specs=[
            pl.BlockSpec((1, ROW_BLK, DT), lambda i: (0, i, 0)),
            pl.BlockSpec((1, ROW_BLK, DT), lambda i: (1, i, 0)),
            pl.BlockSpec((ROW_BLK, DH), lambda i: (i, 0)),
            pl.BlockSpec((1, DH), lambda i: (0, 0)),
            pl.BlockSpec((16, DH), lambda i: (0, 0)),
        ],
        out_specs=pl.BlockSpec((ROW_BLK, DH), lambda i: (i, 0)),
        out_shape=jax.ShapeDtypeStruct((N, DH), jnp.float32),
    )(acc, acc, feature, bias_row, eexp)


def kernel(feature, edge_index, W, a_src, a_dst, bias):
    Wr = W.reshape(IN_DIM, NH, FO)
    Ws = jnp.sum(Wr * a_src[None], axis=-1)
    Wd = jnp.sum(Wr * a_dst[None], axis=-1)
    pad8 = jnp.zeros((IN_DIM, 8), jnp.float32)
    w144 = jnp.concatenate([W, Ws, pad8], axis=1)
    w16 = jnp.concatenate([Wd, pad8], axis=1)

    hext, ud = _project(feature, w144, w16)

    cs = jnp.max(lax.slice(hext, (0, DH), (N, DH + NH)))
    cd = jnp.max(lax.slice(ud, (0, 0), (N, NH)))
    csum = cs + cd
    c = jnp.maximum(csum, 0.2 * csum)
    cvec = jnp.full((16,), c, jnp.float32)

    src = edge_index[0]
    dst = edge_index[1]
    acc = _sc_edges(hext, ud, src, dst, cvec)

    eexp = jnp.repeat(jnp.eye(NH, dtype=jnp.float32), FO, axis=1)
    eexp = jnp.concatenate([eexp, jnp.zeros((16 - NH, DH), jnp.float32)], axis=0)
    bias_row = bias.reshape(1, DH)
    return _epilogue(acc, feature, bias_row, eexp)


# speedup vs baseline: 103.1465x; 1.4399x over previous
import functools

import jax
import jax.numpy as jnp
from jax import lax
from jax.experimental import pallas as pl
from jax.experimental.pallas import tpu as pltpu
from jax.experimental.pallas import tpu_sc as plsc

N = 10000
E = 320000
IN_DIM = 128
NH = 8
FO = 16
DH = NH * FO
DT = DH + 16

NC = 2
NS = 16
NW = NC * NS
EPW = E // NW
CHUNK = 40
NCHUNK = EPW // CHUNK
NPAIR = NCHUNK // 2
NP = 10240
RPT = NP // NS

ROW_BLK = 2000
GRID = N // ROW_BLK


def _proj_body(x_ref, w144_ref, w16_ref, hext_ref, ud_ref):
    x = x_ref[...]
    hext_ref[...] = jnp.dot(x, w144_ref[...], preferred_element_type=jnp.float32)
    ud_ref[...] = jnp.dot(x, w16_ref[...], preferred_element_type=jnp.float32)


def _project(feature, w144, w16):
    return pl.pallas_call(
        _proj_body,
        grid=(GRID,),
        in_specs=[
            pl.BlockSpec((ROW_BLK, IN_DIM), lambda i: (i, 0)),
            pl.BlockSpec((IN_DIM, DT), lambda i: (0, 0)),
            pl.BlockSpec((IN_DIM, 16), lambda i: (0, 0)),
        ],
        out_specs=[
            pl.BlockSpec((ROW_BLK, DT), lambda i: (i, 0)),
            pl.BlockSpec((ROW_BLK, 16), lambda i: (i, 0)),
        ],
        out_shape=[
            jax.ShapeDtypeStruct((N, DT), jnp.float32),
            jax.ShapeDtypeStruct((N, 16), jnp.float32),
        ],
    )(feature, w144, w16)


def _sc_edges_body(hext_hbm, ud_hbm, src_hbm, dst_hbm, cvec_hbm, acc_hbm,
                   idxs_v, idxd_v, idxd_s, hrow_v, udr_v, msg_v, cv_v,
                   acc_sh, semi0, semi1, semg0, semg1, sems0, sems1,
                   semx0, semx1):
    cid = lax.axis_index("c")
    sid = lax.axis_index("s")
    wid = sid * NC + cid
    semi = (semi0, semi1)
    semg = (semg0, semg1)
    sems = (sems0, sems1)
    semx = (semx0, semx1)

    zero16 = jnp.zeros((16,), jnp.float32)

    def zrow(r, carry):
        for pp in range(2):
            for k in range(DT // 16):
                msg_v[pp, r, pl.ds(16 * k, 16)] = zero16
        return carry

    lax.fori_loop(0, CHUNK, zrow, 0)

    def zcopy(j, carry):
        pltpu.sync_copy(msg_v.at[0], acc_sh.at[pl.ds(sid * RPT + j * CHUNK, CHUNK)])
        return carry

    lax.fori_loop(0, RPT // CHUNK, zcopy, 0)
    plsc.subcore_barrier()

    pltpu.sync_copy(cvec_hbm, cv_v)
    cv = cv_v[...]

    def idx_start(x, p):
        base = wid * EPW + x * CHUNK
        pltpu.async_copy(src_hbm.at[pl.ds(base, CHUNK)], idxs_v.at[p], semi[p])
        pltpu.async_copy(dst_hbm.at[pl.ds(base, CHUNK)], idxd_v.at[p], semi[p])

    def idx_wait(p):
        pltpu.make_async_copy(src_hbm.at[pl.ds(0, CHUNK)], idxs_v.at[p], semi[p]).wait()
        pltpu.make_async_copy(dst_hbm.at[pl.ds(0, CHUNK)], idxd_v.at[p], semi[p]).wait()

    def gath_start(p):
        pltpu.async_copy(hext_hbm.at[idxs_v.at[p]], hrow_v.at[p], semg[p])
        pltpu.async_copy(ud_hbm.at[idxd_v.at[p]], udr_v.at[p], semg[p])

    def gath_wait(p):
        pltpu.make_async_copy(hext_hbm.at[idxs_v.at[p]], hrow_v.at[p], semg[p]).wait()
        pltpu.make_async_copy(ud_hbm.at[idxd_v.at[p]], udr_v.at[p], semg[p]).wait()

    def scatidx_start(x, p):
        base = wid * EPW + x * CHUNK
        pltpu.async_copy(dst_hbm.at[pl.ds(base, CHUNK)], idxd_s.at[p], semx[p])

    def scatidx_wait(p):
        pltpu.make_async_copy(dst_hbm.at[pl.ds(0, CHUNK)], idxd_s.at[p], semx[p]).wait()

    def scat_start(p):
        pltpu.async_copy(msg_v.at[p], acc_sh.at[idxd_s.at[p]], sems[p], add=True)

    def scat_wait(p):
        pltpu.make_async_copy(msg_v.at[p], acc_sh.at[idxd_s.at[p]], sems[p]).wait()

    lane = lax.iota(jnp.int32, 16)

    def compute(p):
        def edge2(j, ecarry):
            e0 = 2 * j
            e1 = e0 + 1
            sa = hrow_v[p, e0, pl.ds(DH, 16)] + udr_v[p, e0, :]
            sb = hrow_v[p, e1, pl.ds(DH, 16)] + udr_v[p, e1, :]
            la = jnp.maximum(sa, 0.2 * sa)
            lb = jnp.maximum(sb, 0.2 * sb)
            exa = jnp.exp(la - cv)
            exb = jnp.exp(lb - cv)
            for e, ex in ((e0, exa), (e1, exb)):
                msg_v[p, e, pl.ds(DH, 16)] = ex
                for k in range(NH):
                    onehot = (lane == k).astype(jnp.float32)
                    bk = jnp.full((16,), jnp.sum(ex * onehot))
                    msg_v[p, e, pl.ds(16 * k, 16)] = (
                        hrow_v[p, e, pl.ds(16 * k, 16)] * bk)
            return ecarry

        lax.fori_loop(0, CHUNK // 2, edge2, 0)

    def half(x, p, po):
        idx_wait(po)
        gath_start(po)
        gath_wait(p)
        scat_wait(p)
        scatidx_start(x, p)
        compute(p)
        scatidx_wait(p)
        scat_start(p)
        idx_start(jnp.minimum(x + 2, NCHUNK - 1), p)

    idx_start(0, 0)
    idx_wait(0)
    gath_start(0)
    idx_start(1, 1)
    scatidx_start(0, 0)
    scatidx_start(0, 1)
    scatidx_wait(0)
    scatidx_wait(1)
    scat_start(0)
    scat_start(1)

    def pair(i, carry):
        half(2 * i, 0, 1)
        half(2 * i + 1, 1, 0)
        return carry

    lax.fori_loop(0, NPAIR, pair, 0)
    idx_wait(1)
    gath_wait(0)
    scat_wait(0)
    scat_wait(1)
    plsc.subcore_barrier()

    pltpu.sync_copy(acc_sh.at[pl.ds(sid * RPT, RPT)],
                    acc_hbm.at[cid, pl.ds(sid * RPT, RPT)])


def _sc_edges(hext, ud, src, dst, cvec):
    fn = pl.kernel(
        _sc_edges_body,
        out_type=jax.ShapeDtypeStruct((NC, NP, DT), jnp.float32),
        mesh=plsc.VectorSubcoreMesh(core_axis_name="c", subcore_axis_name="s"),
        scratch_types=[
            pltpu.VMEM((2, CHUNK), jnp.int32),
            pltpu.VMEM((2, CHUNK), jnp.int32),
            pltpu.VMEM((2, CHUNK), jnp.int32),
            pltpu.VMEM((2, CHUNK, DT), jnp.float32),
            pltpu.VMEM((2, CHUNK, 16), jnp.float32),
            pltpu.VMEM((2, CHUNK, DT), jnp.float32),
            pltpu.VMEM((16,), jnp.float32),
            pltpu.VMEM_SHARED((NP, DT), jnp.float32),
            pltpu.SemaphoreType.DMA,
            pltpu.SemaphoreType.DMA,
            pltpu.SemaphoreType.DMA,
            pltpu.SemaphoreType.DMA,
            pltpu.SemaphoreType.DMA,
            pltpu.SemaphoreType.DMA,
            pltpu.SemaphoreType.DMA,
            pltpu.SemaphoreType.DMA,
        ],
        compiler_params=pltpu.CompilerParams(
            needs_layout_passes=False, use_tc_tiling_on_sc=False),
    )
    return fn(hext, ud, src, dst, cvec)


def _epi_body(a0_ref, a1_ref, f_ref, b_ref, e_ref, o_ref):
    s = a0_ref[0] + a1_ref[0]
    num = lax.slice(s, (0, 0), (ROW_BLK, DH))
    den = lax.slice(s, (0, DH), (ROW_BLK, DT))
    den_exp = jnp.dot(den, e_ref[...], preferred_element_type=jnp.float32)
    val = num / (den_exp + 1e-16) + b_ref[...]
    o_ref[...] = f_ref[...] + jnp.where(val > 0, val, jnp.exp(val) - 1.0)


def _epilogue(acc, feature, bias_row, eexp):
    return pl.pallas_call(
        _epi_body,
        grid=(GRID,),
        in_specs=[
            pl.BlockSpec((1, ROW_BLK, DT), lambda i: (0, i, 0)),
            pl.BlockSpec((1, ROW_BLK, DT), lambda i: (1, i, 0)),
            pl.BlockSpec((ROW_BLK, DH), lambda i: (i, 0)),
            pl.BlockSpec((1, DH), lambda i: (0, 0)),
            pl.BlockSpec((16, DH), lambda i: (0, 0)),
        ],
        out_specs=pl.BlockSpec((ROW_BLK, DH), lambda i: (i, 0)),
        out_shape=jax.ShapeDtypeStruct((N, DH), jnp.float32),
    )(acc, acc, feature, bias_row, eexp)


def kernel(feature, edge_index, W, a_src, a_dst, bias):
    Wr = W.reshape(IN_DIM, NH, FO)
    Ws = jnp.sum(Wr * a_src[None], axis=-1)
    Wd = jnp.sum(Wr * a_dst[None], axis=-1)
    pad8 = jnp.zeros((IN_DIM, 8), jnp.float32)
    w144 = jnp.concatenate([W, Ws, pad8], axis=1)
    w16 = jnp.concatenate([Wd, pad8], axis=1)

    hext, ud = _project(feature, w144, w16)

    cs = jnp.max(lax.slice(hext, (0, DH), (N, DH + NH)))
    cd = jnp.max(lax.slice(ud, (0, 0), (N, NH)))
    csum = cs + cd
    c = jnp.maximum(csum, 0.2 * csum)
    cvec = jnp.full((16,), c, jnp.float32)

    src = edge_index[0]
    dst = edge_index[1]
    acc = _sc_edges(hext, ud, src, dst, cvec)

    eexp = jnp.repeat(jnp.eye(NH, dtype=jnp.float32), FO, axis=1)
    eexp = jnp.concatenate([eexp, jnp.zeros((16 - NH, DH), jnp.float32)], axis=0)
    bias_row = bias.reshape(1, DH)
    return _epilogue(acc, feature, bias_row, eexp)
